# baseline (speedup 1.0000x reference)
import jax
import jax.numpy as jnp
from jax import lax
from jax.experimental import pallas as pl
from jax.experimental.pallas import tpu as pltpu
from jax.experimental.pallas import tpu_sc as plsc

ROWS, COLS = 16384, 200
VOCAB, DIM = 6, 4
PACK = 4
GDIM = PACK * DIM
NCOMB = VOCAB ** PACK
NG = ROWS * COLS // PACK
NOUT = ROWS * COLS * DIM

NC, NS = 2, 16
NW = NC * NS
G_STEP = 1280
G_W = NG // NW
STEPS = G_W // G_STEP
TPS = G_STEP * PACK // 128
OSTEP = G_STEP * GDIM

_TCJ = 40
_KC = 512


def _cidx_body(a_ref, c_ref):
    kio = lax.broadcasted_iota(jnp.int32, (_KC, 128), 0)
    mio = lax.broadcasted_iota(jnp.int32, (_KC, 128), 1)
    a = kio % PACK
    wgt = jnp.where(a == 0, 216.0, jnp.where(a == 1, 36.0, jnp.where(a == 2, 6.0, 1.0)))
    m00 = jnp.where(kio // PACK == mio, wgt, 0.0)
    af = a_ref[...].astype(jnp.float32)
    for t in range(ROWS // _KC):
        blk = af[:, _KC * t:_KC * (t + 1)]
        c = jax.lax.dot(blk, m00, preferred_element_type=jnp.float32)
        c_ref[:, t, :] = c.astype(jnp.int32)


def _combined_indices(a):
    return pl.pallas_call(
        _cidx_body,
        out_shape=jax.ShapeDtypeStruct((COLS, ROWS // _KC, 128), jnp.int32),
        grid=(COLS // _TCJ,),
        in_specs=[pl.BlockSpec((_TCJ, ROWS), lambda i: (i, 0))],
        out_specs=pl.BlockSpec((_TCJ, ROWS // _KC, 128), lambda i: (i, 0, 0)),
    )(a)


def _product_table(embed_weight):
    ar = jnp.arange(NCOMB, dtype=jnp.int32)
    digs = jnp.stack(
        [ar // 216 % VOCAB, ar // 36 % VOCAB, ar // VOCAB % VOCAB, ar % VOCAB], axis=1
    )
    onehot = (digs[:, :, None] == jnp.arange(VOCAB)).astype(jnp.float32)
    table = jnp.sum(onehot[:, :, :, None] * embed_weight[None, None, :, :], axis=2)
    return table.reshape(NCOMB, GDIM)


def _gather_body(table_hbm, c_hbm, out_hbm, ttile, cbuf, ob0, ob1, so0, so1):
    wid = lax.axis_index("s") * NC + lax.axis_index("c")
    pltpu.sync_copy(table_hbm, ttile)
    pltpu.sync_copy(c_hbm.at[pl.ds(wid * G_W, G_W)], cbuf)

    iot = lax.iota(jnp.int32, 16)
    pat = iot >> 2
    lm4 = 4 * (iot & 3)
    out_base = wid * G_W * GDIM

    def fill(obuf, sbase):
        def tile(t, cc):
            gb = sbase + 32 * t
            abase = []
            for blk in range(8):
                crep = plsc.load_gather(cbuf, [pat + (gb + 4 * blk)])
                abase.append((crep << 4) + lm4)
            dst0 = 512 * t
            for d in range(DIM):
                dstd = dst0 + 128 * d
                for blk in range(8):
                    v = plsc.load_gather(ttile, [abase[blk] + d])
                    obuf[pl.ds(dstd + 16 * blk, 16)] = v
            return cc

        lax.fori_loop(0, TPS, tile, 0)

    def drain(obuf, sem):
        pltpu.make_async_copy(obuf, out_hbm.at[pl.ds(out_base, OSTEP)], sem).wait()

    def pair(p, cc):
        for half, (obuf, sem) in enumerate(((ob0, so0), (ob1, so1))):
            s = 2 * p + half

            @pl.when(p > 0)
            def _():
                drain(obuf, sem)

            fill(obuf, s * G_STEP)
            pltpu.async_copy(
                obuf, out_hbm.at[pl.ds(out_base + s * OSTEP, OSTEP)], sem
            )
        return cc

    lax.fori_loop(0, STEPS // 2, pair, 0)
    drain(ob0, so0)
    drain(ob1, so1)


def kernel(inputs, embed_weight):
    a = inputs.astype(jnp.int32).T
    table = _product_table(embed_weight.astype(jnp.float32))
    c = _combined_indices(a).reshape(NG)
    mesh = plsc.VectorSubcoreMesh(
        core_axis_name="c", subcore_axis_name="s", num_cores=NC, num_subcores=NS
    )
    out1 = pl.kernel(
        _gather_body,
        out_type=jax.ShapeDtypeStruct((NOUT,), jnp.float32),
        mesh=mesh,
        scratch_types=[
            pltpu.VMEM((NCOMB * GDIM,), jnp.float32),
            pltpu.VMEM((G_W,), jnp.int32),
            pltpu.VMEM((G_STEP * GDIM,), jnp.float32),
            pltpu.VMEM((G_STEP * GDIM,), jnp.float32),
            pltpu.SemaphoreType.DMA,
            pltpu.SemaphoreType.DMA,
        ],
        compiler_params=pltpu.CompilerParams(
            use_tc_tiling_on_sc=False, needs_layout_passes=False
        ),
    )(table.reshape(NCOMB * GDIM), c)
    out4 = out1.reshape(COLS, ROWS // 128, DIM, 128)
    return out4.transpose(1, 3, 0, 2).reshape(ROWS, COLS, DIM)


# speedup vs baseline: 1.0770x; 1.0770x over previous
import jax
import jax.numpy as jnp
from jax import lax
from jax.experimental import pallas as pl
from jax.experimental.pallas import tpu as pltpu
from jax.experimental.pallas import tpu_sc as plsc

ROWS, COLS = 16384, 200
VOCAB, DIM = 6, 4
PACK = 4
GDIM = PACK * DIM
NCOMB = VOCAB ** PACK
NG = ROWS * COLS // PACK
NOUT = ROWS * COLS * DIM

NC, NS = 2, 16
NW = NC * NS
G_STEP = 1280
G_W = NG // NW
STEPS = G_W // G_STEP
TPS = G_STEP * PACK // 128
OSTEP = G_STEP * GDIM
KS = 24
KD = TPS - KS
SG = KS * 32 // 128

_TCJ = 40
_KC = 512


def _cidx_body(a_ref, c_ref):
    kio = lax.broadcasted_iota(jnp.int32, (_KC, 128), 0)
    mio = lax.broadcasted_iota(jnp.int32, (_KC, 128), 1)
    a = kio % PACK
    wgt = jnp.where(a == 0, 216.0, jnp.where(a == 1, 36.0, jnp.where(a == 2, 6.0, 1.0)))
    m00 = jnp.where(kio // PACK == mio, wgt, 0.0)
    af = a_ref[...].astype(jnp.float32)
    for t in range(ROWS // _KC):
        blk = af[:, _KC * t:_KC * (t + 1)]
        c = jax.lax.dot(blk, m00, preferred_element_type=jnp.float32)
        c_ref[:, t, :] = c.astype(jnp.int32)


def _combined_indices(a):
    return pl.pallas_call(
        _cidx_body,
        out_shape=jax.ShapeDtypeStruct((COLS, ROWS // _KC, 128), jnp.int32),
        grid=(COLS // _TCJ,),
        in_specs=[pl.BlockSpec((_TCJ, ROWS), lambda i: (i, 0))],
        out_specs=pl.BlockSpec((_TCJ, ROWS // _KC, 128), lambda i: (i, 0, 0)),
    )(a)


def _product_table(embed_weight):
    ar = jnp.arange(NCOMB, dtype=jnp.int32)
    digs = jnp.stack(
        [ar // 216 % VOCAB, ar // 36 % VOCAB, ar // VOCAB % VOCAB, ar % VOCAB], axis=1
    )
    onehot = (digs[:, :, None] == jnp.arange(VOCAB)).astype(jnp.float32)
    table = jnp.sum(onehot[:, :, :, None] * embed_weight[None, None, :, :], axis=2)
    return table.reshape(NCOMB, GDIM)


def _gather_body(
    table_hbm, c_hbm, out_hbm, tspm, ttile, cbuf, rb0, rb1, ob0, ob1,
    sg0, sg1, so0, so1,
):
    wid = lax.axis_index("s") * NC + lax.axis_index("c")

    @pl.when(lax.axis_index("s") == 0)
    def _():
        pltpu.sync_copy(table_hbm, tspm)

    pltpu.sync_copy(table_hbm, ttile)
    pltpu.sync_copy(c_hbm.at[pl.ds(wid * G_W, G_W)], cbuf)
    plsc.subcore_barrier()

    iot = lax.iota(jnp.int32, 16)
    iot4 = 4 * iot
    pat = iot >> 2
    lm4 = 4 * (iot & 3)
    out_base = wid * G_W * GDIM
    rbufs, gsems = (rb0, rb1), (sg0, sg1)

    def fire(s, b):
        for j in range(SG):
            pltpu.async_copy(
                tspm.at[cbuf.at[pl.ds(s * G_STEP + j * 128, 128)]],
                rbufs[b].at[pl.ds(j * 128, 128)],
                gsems[b],
            )

    def fill(rb, obuf, sbase):
        def ttile_fn(t, cc):
            dst0 = 512 * t
            for d in range(DIM):
                dv = d + iot4
                col = dv & 15
                rbase = (dv >> 4) + 32 * t
                dstd = dst0 + 128 * d
                for blk in range(8):
                    v = plsc.load_gather(rb, [rbase + 4 * blk, col])
                    obuf[pl.ds(dstd + 16 * blk, 16)] = v
            return cc

        lax.fori_loop(0, KS, ttile_fn, 0)

        def dtile(t, cc):
            gb = sbase + 32 * (KS + t)
            creps = [
                plsc.load_gather(cbuf, [pat + (gb + 4 * blk)]) for blk in range(8)
            ]
            dst0 = 512 * (KS + t)
            for d in range(DIM):
                dstd = dst0 + 128 * d
                col = lm4 + d
                for blk in range(8):
                    v = plsc.load_gather(ttile, [creps[blk], col])
                    obuf[pl.ds(dstd + 16 * blk, 16)] = v
            return cc

        lax.fori_loop(0, KD, dtile, 0)

    def drain_out(obuf, sem):
        pltpu.make_async_copy(obuf, out_hbm.at[pl.ds(out_base, OSTEP)], sem).wait()

    def drain_gather(s, b):
        for j in range(SG):
            pltpu.make_async_copy(
                tspm.at[cbuf.at[pl.ds(s * G_STEP + j * 128, 128)]],
                rbufs[b].at[pl.ds(j * 128, 128)],
                gsems[b],
            ).wait()

    fire(0, 0)

    def pair(p, cc):
        for half in range(2):
            s = 2 * p + half
            b = half
            obuf, osem = (ob0, so0) if half == 0 else (ob1, so1)
            if half == 0:
                fire(2 * p + 1, 1)
            else:

                @pl.when(p < STEPS // 2 - 1)
                def _():
                    fire(2 * p + 2, 0)

            drain_gather(s, b)

            @pl.when(p > 0)
            def _():
                drain_out(obuf, osem)

            fill(rbufs[b], obuf, s * G_STEP)
            pltpu.async_copy(
                obuf, out_hbm.at[pl.ds(out_base + s * OSTEP, OSTEP)], osem
            )
        return cc

    lax.fori_loop(0, STEPS // 2, pair, 0)
    drain_out(ob0, so0)
    drain_out(ob1, so1)


def kernel(inputs, embed_weight):
    a = inputs.astype(jnp.int32).T
    table = _product_table(embed_weight.astype(jnp.float32))
    c = _combined_indices(a).reshape(NG)
    mesh = plsc.VectorSubcoreMesh(
        core_axis_name="c", subcore_axis_name="s", num_cores=NC, num_subcores=NS
    )
    out1 = pl.kernel(
        _gather_body,
        out_type=jax.ShapeDtypeStruct((NOUT,), jnp.float32),
        mesh=mesh,
        scratch_types=[
            pltpu.VMEM_SHARED((NCOMB, GDIM), jnp.float32),
            pltpu.VMEM((NCOMB, GDIM), jnp.float32),
            pltpu.VMEM((G_W,), jnp.int32),
            pltpu.VMEM((KS * 32, GDIM), jnp.float32),
            pltpu.VMEM((KS * 32, GDIM), jnp.float32),
            pltpu.VMEM((G_STEP * GDIM,), jnp.float32),
            pltpu.VMEM((G_STEP * GDIM,), jnp.float32),
            pltpu.SemaphoreType.DMA,
            pltpu.SemaphoreType.DMA,
            pltpu.SemaphoreType.DMA,
            pltpu.SemaphoreType.DMA,
        ],
        compiler_params=pltpu.CompilerParams(
            use_tc_tiling_on_sc=False, needs_layout_passes=False
        ),
    )(table, c)
    out4 = out1.reshape(COLS, ROWS // 128, DIM, 128)
    return out4.transpose(1, 3, 0, 2).reshape(ROWS, COLS, DIM)


# speedup vs baseline: 1.1301x; 1.0493x over previous
import jax
import jax.numpy as jnp
from jax import lax
from jax.experimental import pallas as pl
from jax.experimental.pallas import tpu as pltpu
from jax.experimental.pallas import tpu_sc as plsc

ROWS, COLS = 16384, 200
VOCAB, DIM = 6, 4
PACK = 4
GDIM = PACK * DIM
NCOMB = VOCAB ** PACK
NG = ROWS * COLS // PACK
NOUT = ROWS * COLS * DIM

NC, NS = 2, 16
NW = NC * NS
STREAM = 128
CH = 8
G_STEP = CH * STREAM
G_W = NG // NW
STEPS = G_W // G_STEP
C_ROWS = NG // STREAM

_TCJ = 40
_KC = 512


def _cidx_body(a_ref, c_ref):
    kio = lax.broadcasted_iota(jnp.int32, (_KC, 128), 0)
    mio = lax.broadcasted_iota(jnp.int32, (_KC, 128), 1)
    a = kio % PACK
    wgt = jnp.where(a == 0, 216.0, jnp.where(a == 1, 36.0, jnp.where(a == 2, 6.0, 1.0)))
    m00 = jnp.where(kio // PACK == mio, wgt, 0.0)
    af = a_ref[...].astype(jnp.float32)
    for t in range(ROWS // _KC):
        blk = af[:, _KC * t:_KC * (t + 1)]
        c = jax.lax.dot(blk, m00, preferred_element_type=jnp.float32)
        c_ref[:, t, :] = c.astype(jnp.int32)


def _combined_indices(a):
    return pl.pallas_call(
        _cidx_body,
        out_shape=jax.ShapeDtypeStruct((COLS, ROWS // _KC, 128), jnp.int32),
        grid=(COLS // _TCJ,),
        in_specs=[pl.BlockSpec((_TCJ, ROWS), lambda i: (i, 0))],
        out_specs=pl.BlockSpec((_TCJ, ROWS // _KC, 128), lambda i: (i, 0, 0)),
    )(a)


def _product_table(embed_weight):
    ar = jnp.arange(NCOMB, dtype=jnp.int32)
    digs = jnp.stack(
        [ar // 216 % VOCAB, ar // 36 % VOCAB, ar // VOCAB % VOCAB, ar % VOCAB], axis=1
    )
    onehot = (digs[:, :, None] == jnp.arange(VOCAB)).astype(jnp.float32)
    table = jnp.sum(onehot[:, :, :, None] * embed_weight[None, None, :, :], axis=2)
    return table.reshape(NCOMB, GDIM)


def _gather_body(
    table_hbm, c_hbm, out_hbm, tspm, cbuf, rb0, rb1, ob0, ob1, sg0, sg1, so0, so1
):
    wid = lax.axis_index("s") * NC + lax.axis_index("c")

    @pl.when(lax.axis_index("s") == 0)
    def _():
        pltpu.sync_copy(table_hbm, tspm)

    plsc.subcore_barrier()
    pltpu.sync_copy(c_hbm.at[pl.ds(wid * G_W, G_W)], cbuf)

    iot4 = 4 * lax.iota(jnp.int32, 16)
    rbufs, obufs = (rb0, rb1), (ob0, ob1)
    gsems, osems = (sg0, sg1), (so0, so1)
    out_base = wid * G_W * GDIM

    def fire(s):
        b = s % 2
        return [
            pltpu.async_copy(
                tspm.at[cbuf.at[pl.ds(s * G_STEP + j * STREAM, STREAM)]],
                rbufs[b].at[pl.ds(j * STREAM, STREAM)],
                gsems[b],
            )
            for j in range(CH)
        ]

    gathers = {0: fire(0)}
    outcps = {}
    for s in range(STEPS):
        if s + 1 < STEPS:
            gathers[s + 1] = fire(s + 1)
        b = s % 2
        for cp in gathers.pop(s):
            cp.wait()
        if s - 2 in outcps:
            outcps.pop(s - 2).wait()
        rowsbuf, obuf = rbufs[b], obufs[b]

        def tile(q, cc):
            t, d = q >> 2, q & 3
            dv = d + iot4
            col = dv & 15
            rbase = (dv >> 4) + 32 * t
            dst = 512 * t + 128 * d
            for blk in range(8):
                v = plsc.load_gather(rowsbuf, [rbase + 4 * blk, col])
                obuf[pl.ds(dst + 16 * blk, 16)] = v
            return cc

        lax.fori_loop(0, 128, tile, 0)
        outcps[s] = pltpu.async_copy(
            obuf,
            out_hbm.at[pl.ds(out_base + s * G_STEP * GDIM, G_STEP * GDIM)],
            osems[b],
        )
    for cp in outcps.values():
        cp.wait()


def kernel(inputs, embed_weight):
    a = inputs.astype(jnp.int32).T
    table = _product_table(embed_weight.astype(jnp.float32))
    c = _combined_indices(a).reshape(NG)
    mesh = plsc.VectorSubcoreMesh(
        core_axis_name="c", subcore_axis_name="s", num_cores=NC, num_subcores=NS
    )
    out1 = pl.kernel(
        _gather_body,
        out_type=jax.ShapeDtypeStruct((NOUT,), jnp.float32),
        mesh=mesh,
        scratch_types=[
            pltpu.VMEM_SHARED((NCOMB, GDIM), jnp.float32),
            pltpu.VMEM((G_W,), jnp.int32),
            pltpu.VMEM((G_STEP, GDIM), jnp.float32),
            pltpu.VMEM((G_STEP, GDIM), jnp.float32),
            pltpu.VMEM((G_STEP * GDIM,), jnp.float32),
            pltpu.VMEM((G_STEP * GDIM,), jnp.float32),
            pltpu.SemaphoreType.DMA,
            pltpu.SemaphoreType.DMA,
            pltpu.SemaphoreType.DMA,
            pltpu.SemaphoreType.DMA,
        ],
        compiler_params=pltpu.CompilerParams(
            use_tc_tiling_on_sc=False, needs_layout_passes=False
        ),
    )(table, c)
    out4 = out1.reshape(COLS, ROWS // 128, DIM, 128)
    return out4.transpose(1, 3, 0, 2).reshape(ROWS, COLS, DIM)
